# Initial kernel scaffold; baseline (speedup 1.0000x reference)
import jax
import jax.numpy as jnp
from jax.experimental import pallas as pl


def kernel(onset_idx, rhyme_idx, tone_idx, onset_table, rhyme_table, tone_table):
    raise NotImplementedError("write your pallas kernel here")


# speedup vs baseline: 3.0729x; 3.0729x over previous
import jax
import jax.numpy as jnp
from jax.experimental import pallas as pl
from jax.experimental.pallas import tpu as pltpu
from jax.experimental.pallas import tpu_sc as plsc

_WINDOW = 128


def kernel(onset_idx, rhyme_idx, tone_idx, onset_table, rhyme_table, tone_table):
    B, L = onset_idx.shape
    V, D = onset_table.shape
    n = B * L * 3

    table = jnp.concatenate([onset_table, rhyme_table, tone_table], axis=0)
    idx = jnp.stack(
        [
            onset_idx.astype(jnp.int32),
            rhyme_idx.astype(jnp.int32) + V,
            tone_idx.astype(jnp.int32) + 2 * V,
        ],
        axis=-1,
    ).reshape(1, n)

    mesh = plsc.VectorSubcoreMesh(core_axis_name="c", subcore_axis_name="s")

    @pl.kernel(out_type=jax.ShapeDtypeStruct((n, D), jnp.float32), mesh=mesh)
    def gather_kernel(tab_hbm, i_hbm, o_hbm):
        def body(i_vmem, o_vmem):
            pltpu.sync_copy(tab_hbm.at[i_vmem.at[0]], o_vmem)

        pltpu.emit_pipeline(
            body,
            grid=(n // _WINDOW,),
            in_specs=[pl.BlockSpec((1, _WINDOW), index_map=lambda i: (0, i))],
            out_specs=[pl.BlockSpec((_WINDOW, D), index_map=lambda i: (i, 0))],
            core_axis_name=("c", "s"),
            dimension_semantics=(pltpu.PARALLEL,),
        )(i_hbm, o_hbm)

    out = gather_kernel(table, idx)
    return out.reshape(B, L, 3 * D)


# speedup vs baseline: 3.2007x; 1.0416x over previous
import jax
import jax.numpy as jnp
from jax.experimental import pallas as pl
from jax.experimental.pallas import tpu as pltpu
from jax.experimental.pallas import tpu_sc as plsc

_WINDOW = 256


def kernel(onset_idx, rhyme_idx, tone_idx, onset_table, rhyme_table, tone_table):
    B, L = onset_idx.shape
    V, D = onset_table.shape
    n = B * L * 3

    table = jnp.concatenate([onset_table, rhyme_table, tone_table], axis=0)
    idx = jnp.stack(
        [
            onset_idx.astype(jnp.int32),
            rhyme_idx.astype(jnp.int32) + V,
            tone_idx.astype(jnp.int32) + 2 * V,
        ],
        axis=-1,
    ).reshape(1, n)

    mesh = plsc.VectorSubcoreMesh(core_axis_name="c", subcore_axis_name="s")

    @pl.kernel(out_type=jax.ShapeDtypeStruct((n, D), jnp.float32), mesh=mesh)
    def gather_kernel(tab_hbm, i_hbm, o_hbm):
        def body(i_vmem, o_vmem):
            pltpu.sync_copy(tab_hbm.at[i_vmem.at[0]], o_vmem)

        pltpu.emit_pipeline(
            body,
            grid=(n // _WINDOW,),
            in_specs=[pl.BlockSpec((1, _WINDOW), index_map=lambda i: (0, i))],
            out_specs=[pl.BlockSpec((_WINDOW, D), index_map=lambda i: (i, 0))],
            core_axis_name=("c", "s"),
            dimension_semantics=(pltpu.PARALLEL,),
        )(i_hbm, o_hbm)

    out = gather_kernel(table, idx)
    return out.reshape(B, L, 3 * D)


# speedup vs baseline: 3.2057x; 1.0015x over previous
import jax
import jax.numpy as jnp
from jax.experimental import pallas as pl
from jax.experimental.pallas import tpu as pltpu
from jax.experimental.pallas import tpu_sc as plsc

_WINDOW = 384


def kernel(onset_idx, rhyme_idx, tone_idx, onset_table, rhyme_table, tone_table):
    B, L = onset_idx.shape
    V, D = onset_table.shape
    n = B * L * 3

    table = jnp.concatenate([onset_table, rhyme_table, tone_table], axis=0)
    idx = jnp.stack(
        [
            onset_idx.astype(jnp.int32),
            rhyme_idx.astype(jnp.int32) + V,
            tone_idx.astype(jnp.int32) + 2 * V,
        ],
        axis=-1,
    ).reshape(1, n)

    mesh = plsc.VectorSubcoreMesh(core_axis_name="c", subcore_axis_name="s")

    @pl.kernel(out_type=jax.ShapeDtypeStruct((n, D), jnp.float32), mesh=mesh)
    def gather_kernel(tab_hbm, i_hbm, o_hbm):
        def body(i_vmem, o_vmem):
            pltpu.sync_copy(tab_hbm.at[i_vmem.at[0]], o_vmem)

        pltpu.emit_pipeline(
            body,
            grid=(n // _WINDOW,),
            in_specs=[pl.BlockSpec((1, _WINDOW), index_map=lambda i: (0, i))],
            out_specs=[pl.BlockSpec((_WINDOW, D), index_map=lambda i: (i, 0))],
            core_axis_name=("c", "s"),
            dimension_semantics=(pltpu.PARALLEL,),
        )(i_hbm, o_hbm)

    out = gather_kernel(table, idx)
    return out.reshape(B, L, 3 * D)


# speedup vs baseline: 3.7131x; 1.1583x over previous
import jax
import jax.numpy as jnp
from jax.experimental import pallas as pl
from jax.experimental.pallas import tpu as pltpu
from jax.experimental.pallas import tpu_sc as plsc

_WINDOW = 384


def kernel(onset_idx, rhyme_idx, tone_idx, onset_table, rhyme_table, tone_table):
    B, L = onset_idx.shape
    V, D = onset_table.shape
    n = B * L * 3

    table = jnp.concatenate([onset_table, rhyme_table, tone_table], axis=0)
    idx = jnp.stack(
        [
            onset_idx.astype(jnp.int32),
            rhyme_idx.astype(jnp.int32) + V,
            tone_idx.astype(jnp.int32) + 2 * V,
        ],
        axis=-1,
    ).reshape(1, n)

    mesh = plsc.VectorSubcoreMesh(core_axis_name="c", subcore_axis_name="s")

    @pl.kernel(
        out_type=jax.ShapeDtypeStruct((n, D), jnp.float32),
        mesh=mesh,
        scratch_types=[pltpu.VMEM_SHARED((3 * V, D), jnp.float32)],
    )
    def gather_kernel(tab_hbm, i_hbm, o_hbm, tab_spmem):
        sid = jax.lax.axis_index("s")

        @pl.when(sid == 0)
        def _():
            pltpu.sync_copy(tab_hbm, tab_spmem)

        plsc.subcore_barrier()

        def body(i_vmem, o_vmem):
            pltpu.sync_copy(tab_spmem.at[i_vmem.at[0]], o_vmem)

        pltpu.emit_pipeline(
            body,
            grid=(n // _WINDOW,),
            in_specs=[pl.BlockSpec((1, _WINDOW), index_map=lambda i: (0, i))],
            out_specs=[pl.BlockSpec((_WINDOW, D), index_map=lambda i: (i, 0))],
            core_axis_name=("c", "s"),
            dimension_semantics=(pltpu.PARALLEL,),
        )(i_hbm, o_hbm)

    out = gather_kernel(table, idx)
    return out.reshape(B, L, 3 * D)


# speedup vs baseline: 4.2935x; 1.1563x over previous
import jax
import jax.numpy as jnp
from jax.experimental import pallas as pl
from jax.experimental.pallas import tpu as pltpu
from jax.experimental.pallas import tpu_sc as plsc

_WB = 2


def kernel(onset_idx, rhyme_idx, tone_idx, onset_table, rhyme_table, tone_table):
    B, L = onset_idx.shape
    V, D = onset_table.shape
    seg = (L + 7) // 8 * 8
    per_step = _WB * 3 * seg

    table = jnp.concatenate([onset_table, rhyme_table, tone_table], axis=0)
    idx = jnp.stack(
        [
            onset_idx.astype(jnp.int32),
            rhyme_idx.astype(jnp.int32) + V,
            tone_idx.astype(jnp.int32) + 2 * V,
        ],
        axis=1,
    )
    idx = jnp.pad(idx, ((0, 0), (0, 0), (0, seg - L)))
    idx = idx.reshape(B // _WB, 1, per_step)

    mesh = plsc.VectorSubcoreMesh(core_axis_name="c", subcore_axis_name="s")

    @pl.kernel(
        out_type=jax.ShapeDtypeStruct((B, L, 3 * D), jnp.float32),
        mesh=mesh,
        scratch_types=[pltpu.VMEM_SHARED((3 * V, D), jnp.float32)],
    )
    def gather_kernel(tab_hbm, i_hbm, o_hbm, tab_spmem):
        sid = jax.lax.axis_index("s")

        @pl.when(sid == 0)
        def _():
            pltpu.sync_copy(tab_hbm, tab_spmem)

        plsc.subcore_barrier()

        def body(i_vmem, o_vmem):
            iv = i_vmem.at[0, 0]
            for b in range(_WB):
                for t in range(3):
                    pltpu.sync_copy(
                        tab_hbm.at[iv.at[pl.ds((b * 3 + t) * seg, L)]],
                        o_vmem.at[b, :, pl.ds(t * D, D)],
                    )

        pltpu.emit_pipeline(
            body,
            grid=(B // _WB,),
            in_specs=[pl.BlockSpec((1, 1, per_step), index_map=lambda i: (i, 0, 0))],
            out_specs=[pl.BlockSpec((_WB, L, 3 * D), index_map=lambda i: (i, 0, 0))],
            core_axis_name=("c", "s"),
            dimension_semantics=(pltpu.PARALLEL,),
        )(i_hbm, o_hbm)

    return gather_kernel(table, idx)


# speedup vs baseline: 13.1355x; 3.0594x over previous
import jax
import jax.numpy as jnp
from jax.experimental import pallas as pl
from jax.experimental.pallas import tpu as pltpu
from jax.experimental.pallas import tpu_sc as plsc

_W = 384


def kernel(onset_idx, rhyme_idx, tone_idx, onset_table, rhyme_table, tone_table):
    B, L = onset_idx.shape
    V, D = onset_table.shape
    n = B * L * 3

    table = jnp.concatenate([onset_table, rhyme_table, tone_table], axis=0)
    arr = jnp.stack(
        [
            onset_idx.astype(jnp.int32),
            rhyme_idx.astype(jnp.int32) + V,
            tone_idx.astype(jnp.int32) + 2 * V,
        ],
        axis=0,
    )
    idx = (
        arr.reshape(3, B // 8, 8, L)
        .transpose(3, 1, 0, 2)
        .reshape(1, n)
    )

    mesh = plsc.VectorSubcoreMesh(core_axis_name="c", subcore_axis_name="s")

    @pl.kernel(
        out_type=jax.ShapeDtypeStruct((n, D), jnp.float32),
        mesh=mesh,
        scratch_types=[pltpu.VMEM_SHARED((3 * V, D), jnp.float32)],
    )
    def gather_kernel(tab_hbm, i_hbm, o_hbm, tab_spmem):
        sid = jax.lax.axis_index("s")

        @pl.when(sid == 0)
        def _():
            pltpu.sync_copy(tab_hbm, tab_spmem)

        plsc.subcore_barrier()

        def body(i_vmem, o_vmem):
            pltpu.sync_copy(tab_spmem.at[i_vmem.at[0]], o_vmem)

        pltpu.emit_pipeline(
            body,
            grid=(n // _W,),
            in_specs=[pl.BlockSpec((1, _W), index_map=lambda i: (0, i))],
            out_specs=[pl.BlockSpec((_W, D), index_map=lambda i: (i, 0))],
            core_axis_name=("c", "s"),
            dimension_semantics=(pltpu.PARALLEL,),
        )(i_hbm, o_hbm)

    rows = gather_kernel(table, idx)
    out = (
        rows.reshape(L, B // 8, 3, 8, D)
        .transpose(1, 3, 0, 2, 4)
        .reshape(B, L, 3 * D)
    )
    return out


# speedup vs baseline: 17.8162x; 1.3563x over previous
import dataclasses

import jax
import jax.numpy as jnp
from jax import lax
from jax.experimental import pallas as pl
from jax.experimental.pallas import tpu as pltpu
from jax.experimental.pallas import tpu_sc as plsc

_NW = 32


def kernel(onset_idx, rhyme_idx, tone_idx, onset_table, rhyme_table, tone_table):
    B, L = onset_idx.shape
    V, D = onset_table.shape
    n = B * L * 3
    bw = B // _NW
    win = (bw // 8) * 24

    mesh = plsc.VectorSubcoreMesh(core_axis_name="c", subcore_axis_name="s")
    cp = pltpu.CompilerParams()
    if "needs_layout_passes" in pltpu.CompilerParams.__dataclass_fields__:
        cp = dataclasses.replace(cp, needs_layout_passes=False)

    @pl.kernel(
        out_type=jax.ShapeDtypeStruct((n, D), jnp.float32),
        mesh=mesh,
        compiler_params=cp,
        scratch_types=[
            pltpu.VMEM_SHARED((3 * V, D), jnp.float32),
            pltpu.VMEM((bw, L), jnp.int32),
            pltpu.VMEM((L * win,), jnp.int32),
            pltpu.VMEM((win // 2, D), jnp.float32),
            pltpu.VMEM((win // 2, D), jnp.float32),
            pltpu.SemaphoreType.DMA,
            pltpu.SemaphoreType.DMA,
        ],
    )
    def gather_kernel(
        i1_hbm, i2_hbm, i3_hbm, t1_hbm, t2_hbm, t3_hbm, o_hbm,
        tab_spmem, raw_v, perm_v, buf0, buf1, sem0, sem1,
    ):
        cid = lax.axis_index("c")
        sid = lax.axis_index("s")
        wid = cid * 16 + sid
        b0 = wid * bw

        @pl.when(sid == 0)
        def _():
            pltpu.sync_copy(t1_hbm, tab_spmem.at[pl.ds(0, V)])
            pltpu.sync_copy(t2_hbm, tab_spmem.at[pl.ds(V, V)])
            pltpu.sync_copy(t3_hbm, tab_spmem.at[pl.ds(2 * V, V)])

        lane = lax.iota(jnp.int32, 16)
        dst_pat = (lane // 8) * 24 + (lane % 8)
        for t, i_hbm in enumerate((i1_hbm, i2_hbm, i3_hbm)):
            pltpu.sync_copy(i_hbm.at[pl.ds(b0, bw)], raw_v)

            @pl.loop(0, L)
            def _(l):
                for j in range(bw // 16):
                    rows = j * 16 + lane
                    cols = jnp.full((16,), l, jnp.int32)
                    v = plsc.load_gather(raw_v, [rows, cols]) + t * V
                    dst = l * win + j * 48 + t * 8 + dst_pat
                    plsc.store_scatter(perm_v, [dst], v)

        plsc.subcore_barrier()

        half = win // 2

        def do_step(s, buf, sem):
            @pl.when(s >= 2)
            def _():
                pltpu.make_async_copy(buf, o_hbm.at[pl.ds(0, half)], sem).wait()

            pltpu.sync_copy(tab_spmem.at[perm_v.at[pl.ds(s * half, half)]], buf)
            p0 = (s // 2) * (B // 8) * 24 + wid * win + (s % 2) * half
            pltpu.make_async_copy(buf, o_hbm.at[pl.ds(p0, half)], sem).start()

        @pl.loop(0, 2 * L, step=2)
        def _(s):
            do_step(s, buf0, sem0)
            do_step(s + 1, buf1, sem1)

        pltpu.make_async_copy(buf0, o_hbm.at[pl.ds(0, half)], sem0).wait()
        pltpu.make_async_copy(buf1, o_hbm.at[pl.ds(0, half)], sem1).wait()

    rows = gather_kernel(
        onset_idx.astype(jnp.int32),
        rhyme_idx.astype(jnp.int32),
        tone_idx.astype(jnp.int32),
        onset_table,
        rhyme_table,
        tone_table,
    )
    out = (
        rows.reshape(L, B // 8, 3, 8, D)
        .transpose(1, 3, 0, 2, 4)
        .reshape(B, L, 3 * D)
    )
    return out


# speedup vs baseline: 18.1694x; 1.0198x over previous
import dataclasses

import jax
import jax.numpy as jnp
from jax import lax
from jax.experimental import pallas as pl
from jax.experimental.pallas import tpu as pltpu
from jax.experimental.pallas import tpu_sc as plsc

_NW = 32


def kernel(onset_idx, rhyme_idx, tone_idx, onset_table, rhyme_table, tone_table):
    B, L = onset_idx.shape
    V, D = onset_table.shape
    n = B * L * 3
    bw = B // _NW
    win = (bw // 8) * 24

    mesh = plsc.VectorSubcoreMesh(core_axis_name="c", subcore_axis_name="s")
    cp = pltpu.CompilerParams()
    if "needs_layout_passes" in pltpu.CompilerParams.__dataclass_fields__:
        cp = dataclasses.replace(cp, needs_layout_passes=False)

    @pl.kernel(
        out_type=jax.ShapeDtypeStruct((n, D), jnp.float32),
        mesh=mesh,
        compiler_params=cp,
        scratch_types=[
            pltpu.VMEM_SHARED((3 * V, D), jnp.float32),
            pltpu.VMEM((bw, L), jnp.int32),
            pltpu.VMEM((L * win,), jnp.int32),
            pltpu.VMEM((win // 4, D), jnp.float32),
            pltpu.VMEM((win // 4, D), jnp.float32),
            pltpu.VMEM((win // 4, D), jnp.float32),
            pltpu.VMEM((win // 4, D), jnp.float32),
            pltpu.SemaphoreType.DMA,
            pltpu.SemaphoreType.DMA,
            pltpu.SemaphoreType.DMA,
            pltpu.SemaphoreType.DMA,
            pltpu.SemaphoreType.DMA,
            pltpu.SemaphoreType.DMA,
            pltpu.SemaphoreType.DMA,
            pltpu.SemaphoreType.DMA,
        ],
    )
    def gather_kernel(
        i1_hbm, i2_hbm, i3_hbm, t1_hbm, t2_hbm, t3_hbm, o_hbm,
        tab_spmem, raw_v, perm_v, buf0, buf1, buf2, buf3,
        gsem0, gsem1, gsem2, gsem3, wsem0, wsem1, wsem2, wsem3,
    ):
        cid = lax.axis_index("c")
        sid = lax.axis_index("s")
        wid = cid * 16 + sid
        b0 = wid * bw

        @pl.when(sid == 0)
        def _():
            pltpu.sync_copy(t1_hbm, tab_spmem.at[pl.ds(0, V)])
            pltpu.sync_copy(t2_hbm, tab_spmem.at[pl.ds(V, V)])
            pltpu.sync_copy(t3_hbm, tab_spmem.at[pl.ds(2 * V, V)])

        lane = lax.iota(jnp.int32, 16)
        dst_pat = (lane // 8) * 24 + (lane % 8)
        for t, i_hbm in enumerate((i1_hbm, i2_hbm, i3_hbm)):
            pltpu.sync_copy(i_hbm.at[pl.ds(b0, bw)], raw_v)

            @pl.loop(0, L)
            def _(l):
                for j in range(bw // 16):
                    rows = j * 16 + lane
                    cols = jnp.full((16,), l, jnp.int32)
                    v = plsc.load_gather(raw_v, [rows, cols]) + t * V
                    dst = l * win + j * 48 + t * 8 + dst_pat
                    plsc.store_scatter(perm_v, [dst], v)

        plsc.subcore_barrier()

        q = win // 4
        bufs = (buf0, buf1, buf2, buf3)
        gsems = (gsem0, gsem1, gsem2, gsem3)
        wsems = (wsem0, wsem1, wsem2, wsem3)
        nsteps = 4 * L

        def gstart(s, buf, gsem):
            pltpu.make_async_copy(
                tab_spmem.at[perm_v.at[pl.ds(s * q, q)]], buf, gsem
            ).start()

        def gwait(buf, gsem):
            pltpu.make_async_copy(
                tab_spmem.at[perm_v.at[pl.ds(0, q)]], buf, gsem
            ).wait()

        def wstart(s, buf, wsem):
            p0 = (s // 4) * (B // 8) * 24 + wid * win + (s % 4) * q
            pltpu.make_async_copy(buf, o_hbm.at[pl.ds(p0, q)], wsem).start()

        def wwait(buf, wsem):
            pltpu.make_async_copy(buf, o_hbm.at[pl.ds(0, q)], wsem).wait()

        for k in range(4):
            gstart(k, bufs[k], gsems[k])

        @pl.loop(0, nsteps, step=4)
        def _(s):
            for k in range(4):
                gwait(bufs[k], gsems[k])
                wstart(s + k, bufs[k], wsems[k])
            for k in range(4):

                @pl.when(s + 4 + k < nsteps)
                def _(k=k):
                    wwait(bufs[k], wsems[k])
                    gstart(s + 4 + k, bufs[k], gsems[k])

        for k in range(4):
            wwait(bufs[k], wsems[k])

    rows = gather_kernel(
        onset_idx.astype(jnp.int32),
        rhyme_idx.astype(jnp.int32),
        tone_idx.astype(jnp.int32),
        onset_table,
        rhyme_table,
        tone_table,
    )
    out = (
        rows.reshape(L, B // 8, 3, 8, D)
        .transpose(1, 3, 0, 2, 4)
        .reshape(B, L, 3 * D)
    )
    return out


# speedup vs baseline: 18.5057x; 1.0185x over previous
import dataclasses

import jax
import jax.numpy as jnp
from jax import lax
from jax.experimental import pallas as pl
from jax.experimental.pallas import tpu as pltpu
from jax.experimental.pallas import tpu_sc as plsc

_NW = 32


def kernel(onset_idx, rhyme_idx, tone_idx, onset_table, rhyme_table, tone_table):
    B, L = onset_idx.shape
    V, D = onset_table.shape
    n = B * L * 3
    bw = B // _NW
    win = (bw // 8) * 24

    mesh = plsc.VectorSubcoreMesh(core_axis_name="c", subcore_axis_name="s")
    cp = pltpu.CompilerParams()
    if "needs_layout_passes" in pltpu.CompilerParams.__dataclass_fields__:
        cp = dataclasses.replace(cp, needs_layout_passes=False)

    @pl.kernel(
        out_type=jax.ShapeDtypeStruct((n, D), jnp.float32),
        mesh=mesh,
        compiler_params=cp,
        scratch_types=[
            pltpu.VMEM_SHARED((3 * V, D), jnp.float32),
            pltpu.VMEM((bw, L), jnp.int32),
            pltpu.VMEM((L * win,), jnp.int32),
            pltpu.VMEM((win // 4, D), jnp.float32),
            pltpu.VMEM((win // 4, D), jnp.float32),
            pltpu.VMEM((win // 4, D), jnp.float32),
            pltpu.VMEM((win // 4, D), jnp.float32),
            pltpu.SemaphoreType.DMA,
            pltpu.SemaphoreType.DMA,
            pltpu.SemaphoreType.DMA,
            pltpu.SemaphoreType.DMA,
            pltpu.SemaphoreType.DMA,
            pltpu.SemaphoreType.DMA,
            pltpu.SemaphoreType.DMA,
            pltpu.SemaphoreType.DMA,
        ],
    )
    def gather_kernel(
        i1_hbm, i2_hbm, i3_hbm, t1_hbm, t2_hbm, t3_hbm, o_hbm,
        tab_spmem, raw_v, perm_v, buf0, buf1, buf2, buf3,
        gsem0, gsem1, gsem2, gsem3, wsem0, wsem1, wsem2, wsem3,
    ):
        cid = lax.axis_index("c")
        sid = lax.axis_index("s")
        wid = cid * 16 + sid
        b0 = wid * bw

        for t, t_hbm in enumerate((t1_hbm, t2_hbm, t3_hbm)):

            @pl.when(sid == t)
            def _(t=t, t_hbm=t_hbm):
                pltpu.make_async_copy(
                    t_hbm, tab_spmem.at[pl.ds(t * V, V)], gsem0
                ).start()

        lane = lax.iota(jnp.int32, 16)
        dst_pat = (lane // 8) * 24 + (lane % 8)
        for t, i_hbm in enumerate((i1_hbm, i2_hbm, i3_hbm)):
            pltpu.sync_copy(i_hbm.at[pl.ds(b0, bw)], raw_v)

            @pl.loop(0, L)
            def _(l):
                for j in range(bw // 16):
                    rows = j * 16 + lane
                    cols = jnp.full((16,), l, jnp.int32)
                    v = plsc.load_gather(raw_v, [rows, cols]) + t * V
                    dst = l * win + j * 48 + t * 8 + dst_pat
                    plsc.store_scatter(perm_v, [dst], v)

        @pl.when(sid < 3)
        def _():
            pltpu.make_async_copy(
                t1_hbm, tab_spmem.at[pl.ds(0, V)], gsem0
            ).wait()

        plsc.subcore_barrier()

        q = win // 4
        bufs = (buf0, buf1, buf2, buf3)
        gsems = (gsem0, gsem1, gsem2, gsem3)
        wsems = (wsem0, wsem1, wsem2, wsem3)
        nsteps = 4 * L

        def gstart(s, buf, gsem):
            pltpu.make_async_copy(
                tab_spmem.at[perm_v.at[pl.ds(s * q, q)]], buf, gsem
            ).start()

        def gwait(buf, gsem):
            pltpu.make_async_copy(
                tab_spmem.at[perm_v.at[pl.ds(0, q)]], buf, gsem
            ).wait()

        def wstart(s, buf, wsem):
            p0 = (s // 4) * (B // 8) * 24 + wid * win + (s % 4) * q
            pltpu.make_async_copy(buf, o_hbm.at[pl.ds(p0, q)], wsem).start()

        def wwait(buf, wsem):
            pltpu.make_async_copy(buf, o_hbm.at[pl.ds(0, q)], wsem).wait()

        for k in range(4):
            gstart(k, bufs[k], gsems[k])

        @pl.loop(0, nsteps, step=4)
        def _(s):
            for k in range(4):
                gwait(bufs[k], gsems[k])
                wstart(s + k, bufs[k], wsems[k])
            for k in range(4):

                @pl.when(s + 4 + k < nsteps)
                def _(k=k):
                    wwait(bufs[k], wsems[k])
                    gstart(s + 4 + k, bufs[k], gsems[k])

        for k in range(4):
            wwait(bufs[k], wsems[k])

    rows = gather_kernel(
        onset_idx.astype(jnp.int32),
        rhyme_idx.astype(jnp.int32),
        tone_idx.astype(jnp.int32),
        onset_table,
        rhyme_table,
        tone_table,
    )
    out = (
        rows.reshape(L, B // 8, 3, 8, D)
        .transpose(1, 3, 0, 2, 4)
        .reshape(B, L, 3 * D)
    )
    return out
